# baseline (speedup 1.0000x reference)
import functools

import jax
import jax.numpy as jnp
import numpy as np
from jax import lax
from jax.experimental import pallas as pl
from jax.experimental.pallas import tpu as pltpu
from jax.experimental.pallas import tpu_sc as plsc

B = 16384
L = 200
EMB = 32
NUM_CLASSES = 10

NC = 2
NS = 16
NW = NC * NS
BPW = B // NW
RPW = BPW * L
RB = 8
CHUNK = RB * L
GCH = 128
SUB = 13
OVL = SUB * GCH - CHUNK
NCH = BPW // RB
ACC_ROWS = NS * BPW
ACC_PAD = 8

def _make_seg(bpw):
    nch = bpw // RB
    acc_rows = NS * bpw
    full = (np.arange(NS * bpw * L, dtype=np.int32) // L).reshape(NS, nch, CHUNK)
    sega = full[:, :, : 12 * GCH].reshape(NS, nch, 12, GCH)
    segl = np.concatenate(
        [np.full((NS, nch, 1, OVL), acc_rows, dtype=np.int32),
         full[:, :, 12 * GCH:].reshape(NS, nch, 1, GCH - OVL)], axis=-1)
    return np.concatenate([sega, segl], axis=2).reshape(NS * nch * SUB, GCH)


_SEG = _make_seg(BPW)
_ZROWS = np.zeros((BPW, EMB), dtype=np.float32)
HALF_BPW = BPW // 2
_SEG_H = _make_seg(HALF_BPW)
_ZROWS_H = np.zeros((HALF_BPW, EMB), dtype=np.float32)


def _pool_sc(x, table, seg, zrows, bpw):
    mesh = plsc.VectorSubcoreMesh(core_axis_name="c", subcore_axis_name="s")
    nb = x.shape[0]
    nch = bpw // RB
    acc_rows = NS * bpw

    @functools.partial(
        pl.kernel,
        out_type=jax.ShapeDtypeStruct((nb, EMB), jnp.float32),
        mesh=mesh,
        scratch_types=[
            pltpu.VMEM((2, CHUNK), jnp.int32),
            pltpu.VMEM((2, SUB, GCH), jnp.int32),
            pltpu.VMEM((2, SUB, GCH, EMB), jnp.float32),
            pltpu.VMEM_SHARED((acc_rows + ACC_PAD, EMB), jnp.float32),
            pltpu.SemaphoreType.DMA((2,)),
            pltpu.SemaphoreType.DMA((2,)),
            pltpu.SemaphoreType.DMA((2, SUB)),
        ],
        compiler_params=pltpu.CompilerParams(use_tc_tiling_on_sc=False, skip_device_barrier=True),
    )
    def k(x_hbm, tab_hbm, seg_hbm, z_hbm, out_hbm,
          idx_v, seg_v, rows_v, acc_sh, isem, ssem, gsem):
        sid = lax.axis_index("s")
        wid = lax.axis_index("c") * NS + sid
        base_row = wid * bpw
        segbase = sid * (nch * SUB)

        pltpu.sync_copy(z_hbm, acc_sh.at[pl.ds(sid * bpw, bpw)])

        def load_idx(c, b):
            for r in range(RB):
                pltpu.async_copy(x_hbm.at[base_row + c * RB + r],
                                 idx_v.at[b, pl.ds(r * L, L)], isem.at[b])
            pltpu.async_copy(seg_hbm.at[pl.ds(segbase + c * SUB, SUB)],
                             seg_v.at[b], ssem.at[b])

        def wait_idx(b):
            for r in range(RB):
                pltpu.make_async_copy(x_hbm.at[0],
                                      idx_v.at[b, pl.ds(r * L, L)],
                                      isem.at[b]).wait()
            pltpu.make_async_copy(seg_hbm.at[pl.ds(0, SUB)],
                                  seg_v.at[b], ssem.at[b]).wait()

        def fire_gathers(b):
            for g in range(SUB):
                off = min(g * GCH, CHUNK - GCH)
                pltpu.async_copy(
                    tab_hbm.at[idx_v.at[b, pl.ds(off, GCH)]],
                    rows_v.at[b, g],
                    gsem.at[b, g],
                )

        def wait_gathers(b):
            for g in range(SUB):
                pltpu.make_async_copy(tab_hbm.at[pl.ds(0, GCH)],
                                      rows_v.at[b, g], gsem.at[b, g]).wait()

        load_idx(0, 0)
        wait_idx(0)
        fire_gathers(0)
        load_idx(1, 1)

        @pl.loop(0, nch, step=2)
        def _block(c0):
            for b in range(2):
                c = c0 + b
                nb = 1 - b

                @pl.when(c + 1 < nch)
                def _():
                    wait_idx(nb)
                    fire_gathers(nb)

                wait_gathers(b)
                for g in range(SUB):
                    pltpu.sync_copy(rows_v.at[b, g], acc_sh.at[seg_v.at[b, g]],
                                    add=True)

                @pl.when(c + 2 < nch)
                def _():
                    load_idx(c + 2, b)

        pltpu.sync_copy(acc_sh.at[pl.ds(sid * bpw, bpw)],
                        out_hbm.at[pl.ds(wid * bpw, bpw)])

    return k(x, table, seg, zrows)


def _mlp_tc(pooled_sum, w1t, b1, w2t, b2, w3t, b3):
    def body(p_ref, w1_ref, b1_ref, w2_ref, b2_ref, w3_ref, b3_ref, o_ref):
        p = p_ref[...] * (1.0 / L)
        h = jnp.dot(p, w1_ref[...], precision=lax.Precision.HIGHEST,
                    preferred_element_type=jnp.float32) + b1_ref[...]
        h = jnp.maximum(h, 0.0)
        h = jnp.dot(h, w2_ref[...], precision=lax.Precision.HIGHEST,
                    preferred_element_type=jnp.float32) + b2_ref[...]
        h = jnp.maximum(h, 0.0)
        o_ref[...] = jnp.dot(h, w3_ref[...], precision=lax.Precision.HIGHEST,
                             preferred_element_type=jnp.float32) + b3_ref[...]

    BB = 2048
    nb = pooled_sum.shape[0]
    return pl.pallas_call(
        body,
        grid=(nb // BB,),
        in_specs=[
            pl.BlockSpec((BB, EMB), lambda i: (i, 0)),
            pl.BlockSpec(w1t.shape, lambda i: (0, 0)),
            pl.BlockSpec(b1.shape, lambda i: (0, 0)),
            pl.BlockSpec(w2t.shape, lambda i: (0, 0)),
            pl.BlockSpec(b2.shape, lambda i: (0, 0)),
            pl.BlockSpec(w3t.shape, lambda i: (0, 0)),
            pl.BlockSpec(b3.shape, lambda i: (0, 0)),
        ],
        out_specs=pl.BlockSpec((BB, NUM_CLASSES), lambda i: (i, 0)),
        out_shape=jax.ShapeDtypeStruct((nb, NUM_CLASSES), jnp.float32),
    )(pooled_sum, w1t, b1, w2t, b2, w3t, b3)


def kernel(x, table, W1, b1, W2, b2, W3, b3):
    wargs = (W1.T, b1.reshape(1, -1), W2.T, b2.reshape(1, -1),
             W3.T, b3.reshape(1, -1))
    half = B // 2
    p_lo = _pool_sc(x[:half], table, _SEG_H, _ZROWS_H, HALF_BPW)
    p_hi = _pool_sc(x[half:], table, _SEG_H, _ZROWS_H, HALF_BPW)
    o_lo = _mlp_tc(p_lo, *wargs)
    o_hi = _mlp_tc(p_hi, *wargs)
    return jnp.concatenate([o_lo, o_hi], axis=0)


# speedup vs baseline: 1.0006x; 1.0006x over previous
import functools

import jax
import jax.numpy as jnp
import numpy as np
from jax import lax
from jax.experimental import pallas as pl
from jax.experimental.pallas import tpu as pltpu
from jax.experimental.pallas import tpu_sc as plsc

B = 16384
L = 200
EMB = 32
NUM_CLASSES = 10

NC = 2
NS = 16
NW = NC * NS
BPW = B // NW
RPW = BPW * L
RB = 8
CHUNK = RB * L
GCH = 128
SUB = 13
OVL = SUB * GCH - CHUNK
NCH = BPW // RB
ACC_ROWS = NS * BPW
ACC_PAD = 8

def _make_seg(bpw):
    nch = bpw // RB
    acc_rows = NS * bpw
    full = (np.arange(NS * bpw * L, dtype=np.int32) // L).reshape(NS, nch, CHUNK)
    sega = full[:, :, : 12 * GCH].reshape(NS, nch, 12, GCH)
    segl = np.concatenate(
        [np.full((NS, nch, 1, OVL), acc_rows, dtype=np.int32),
         full[:, :, 12 * GCH:].reshape(NS, nch, 1, GCH - OVL)], axis=-1)
    return np.concatenate([sega, segl], axis=2).reshape(NS * nch * SUB, GCH)


_SEG = _make_seg(BPW)
_ZROWS = np.zeros((BPW, EMB), dtype=np.float32)
HALF_BPW = BPW // 2
_SEG_H = _make_seg(HALF_BPW)
_ZROWS_H = np.zeros((HALF_BPW, EMB), dtype=np.float32)


def _pool_sc(x, table, seg, zrows, bpw):
    mesh = plsc.VectorSubcoreMesh(core_axis_name="c", subcore_axis_name="s")
    nb = x.shape[0]
    nch = bpw // RB
    acc_rows = NS * bpw

    @functools.partial(
        pl.kernel,
        out_type=jax.ShapeDtypeStruct((nb, EMB), jnp.float32),
        mesh=mesh,
        scratch_types=[
            pltpu.VMEM((2, CHUNK), jnp.int32),
            pltpu.VMEM((2, SUB, GCH), jnp.int32),
            pltpu.VMEM((2, SUB, GCH, EMB), jnp.float32),
            pltpu.VMEM_SHARED((acc_rows + ACC_PAD, EMB), jnp.float32),
            pltpu.SemaphoreType.DMA((2,)),
            pltpu.SemaphoreType.DMA((2,)),
            pltpu.SemaphoreType.DMA((2, SUB)),
        ],
        compiler_params=pltpu.CompilerParams(use_tc_tiling_on_sc=False),
    )
    def k(x_hbm, tab_hbm, seg_hbm, z_hbm, out_hbm,
          idx_v, seg_v, rows_v, acc_sh, isem, ssem, gsem):
        sid = lax.axis_index("s")
        wid = lax.axis_index("c") * NS + sid
        base_row = wid * bpw
        segbase = sid * (nch * SUB)

        pltpu.sync_copy(z_hbm, acc_sh.at[pl.ds(sid * bpw, bpw)])

        def load_idx(c, b):
            for r in range(RB):
                pltpu.async_copy(x_hbm.at[base_row + c * RB + r],
                                 idx_v.at[b, pl.ds(r * L, L)], isem.at[b])
            pltpu.async_copy(seg_hbm.at[pl.ds(segbase + c * SUB, SUB)],
                             seg_v.at[b], ssem.at[b])

        def wait_idx(b):
            for r in range(RB):
                pltpu.make_async_copy(x_hbm.at[0],
                                      idx_v.at[b, pl.ds(r * L, L)],
                                      isem.at[b]).wait()
            pltpu.make_async_copy(seg_hbm.at[pl.ds(0, SUB)],
                                  seg_v.at[b], ssem.at[b]).wait()

        def fire_gathers(b):
            for g in range(SUB):
                off = min(g * GCH, CHUNK - GCH)
                pltpu.async_copy(
                    tab_hbm.at[idx_v.at[b, pl.ds(off, GCH)]],
                    rows_v.at[b, g],
                    gsem.at[b, g],
                )

        def wait_gathers(b):
            for g in range(SUB):
                pltpu.make_async_copy(tab_hbm.at[pl.ds(0, GCH)],
                                      rows_v.at[b, g], gsem.at[b, g]).wait()

        load_idx(0, 0)
        wait_idx(0)
        fire_gathers(0)
        load_idx(1, 1)

        @pl.loop(0, nch, step=2)
        def _block(c0):
            for b in range(2):
                c = c0 + b
                nb = 1 - b

                @pl.when(c + 1 < nch)
                def _():
                    wait_idx(nb)
                    fire_gathers(nb)

                wait_gathers(b)
                for g in range(SUB):
                    pltpu.sync_copy(rows_v.at[b, g], acc_sh.at[seg_v.at[b, g]],
                                    add=True)

                @pl.when(c + 2 < nch)
                def _():
                    load_idx(c + 2, b)

        pltpu.sync_copy(acc_sh.at[pl.ds(sid * bpw, bpw)],
                        out_hbm.at[pl.ds(wid * bpw, bpw)])

    return k(x, table, seg, zrows)


def _mlp_tc(pooled_sum, w1t, b1, w2t, b2, w3t, b3):
    def body(p_ref, w1_ref, b1_ref, w2_ref, b2_ref, w3_ref, b3_ref, o_ref):
        p = p_ref[...] * (1.0 / L)
        h = jnp.dot(p, w1_ref[...], precision=lax.Precision.HIGHEST,
                    preferred_element_type=jnp.float32) + b1_ref[...]
        h = jnp.maximum(h, 0.0)
        h = jnp.dot(h, w2_ref[...], precision=lax.Precision.HIGHEST,
                    preferred_element_type=jnp.float32) + b2_ref[...]
        h = jnp.maximum(h, 0.0)
        o_ref[...] = jnp.dot(h, w3_ref[...], precision=lax.Precision.HIGHEST,
                             preferred_element_type=jnp.float32) + b3_ref[...]

    BB = 2048
    nb = pooled_sum.shape[0]
    return pl.pallas_call(
        body,
        grid=(nb // BB,),
        in_specs=[
            pl.BlockSpec((BB, EMB), lambda i: (i, 0)),
            pl.BlockSpec(w1t.shape, lambda i: (0, 0)),
            pl.BlockSpec(b1.shape, lambda i: (0, 0)),
            pl.BlockSpec(w2t.shape, lambda i: (0, 0)),
            pl.BlockSpec(b2.shape, lambda i: (0, 0)),
            pl.BlockSpec(w3t.shape, lambda i: (0, 0)),
            pl.BlockSpec(b3.shape, lambda i: (0, 0)),
        ],
        out_specs=pl.BlockSpec((BB, NUM_CLASSES), lambda i: (i, 0)),
        out_shape=jax.ShapeDtypeStruct((nb, NUM_CLASSES), jnp.float32),
    )(pooled_sum, w1t, b1, w2t, b2, w3t, b3)


def kernel(x, table, W1, b1, W2, b2, W3, b3):
    wargs = (W1.T, b1.reshape(1, -1), W2.T, b2.reshape(1, -1),
             W3.T, b3.reshape(1, -1))
    half = B // 2
    p_lo = _pool_sc(x[:half], table, _SEG_H, _ZROWS_H, HALF_BPW)
    p_hi = _pool_sc(x[half:], table, _SEG_H, _ZROWS_H, HALF_BPW)
    o_lo = _mlp_tc(p_lo, *wargs)
    o_hi = _mlp_tc(p_hi, *wargs)
    return jnp.concatenate([o_lo, o_hi], axis=0)
